# baseline (speedup 1.0000x reference)
import functools

import jax
import jax.numpy as jnp
from jax import lax
from jax.experimental import pallas as pl
from jax.experimental.pallas import tpu as pltpu
from jax.experimental.pallas import tpu_sc as plsc

_G = 100
_K = 8


def _sc_lookup(src_table, tgt_table, src_idx, tgt_idx, B, L):
    V, D = src_table.shape
    NW, n_g, _ = src_idx.shape
    per_w = B // NW
    n_outer = n_g // _K
    epg = _G // L

    mesh = plsc.VectorSubcoreMesh(core_axis_name="c", subcore_axis_name="s")
    info = plsc.get_sparse_core_info()
    NC = info.num_cores

    @functools.partial(
        pl.kernel,
        mesh=mesh,
        out_type=[
            jax.ShapeDtypeStruct((B, L, D), jnp.float32),
            jax.ShapeDtypeStruct((B, L, D), jnp.float32),
        ],
        scratch_types=[
            pltpu.VMEM((n_g, _G), jnp.int32),
            pltpu.VMEM((n_g, _G), jnp.int32),
            pltpu.VMEM((_K * _G, D), jnp.float32),
            pltpu.SemaphoreType.DMA((_K,)),
            pltpu.SemaphoreType.DMA((_K * epg,)),
        ],
    )
    def k(src_t, tgt_t, src_i, tgt_i, src_o, tgt_o,
          idx_src_v, idx_tgt_v, rows_v, sem_g, sem_w):
        wid = lax.axis_index("s") * NC + lax.axis_index("c")
        base_e = wid * per_w

        pltpu.sync_copy(src_i.at[wid], idx_src_v)
        pltpu.sync_copy(tgt_i.at[wid], idx_tgt_v)

        def wb_wait(out_hbm, j):
            pltpu.make_async_copy(
                rows_v.at[pl.ds(0, L)], out_hbm.at[base_e], sem_w.at[j]
            ).wait()

        def side(table, idx_v, out_hbm, prev_out):
            def outer(t, carry):
                gathers = []
                for b in range(_K):
                    @pl.when(t > 0)
                    def _():
                        for e in range(epg):
                            wb_wait(out_hbm, b * epg + e)

                    if prev_out is not None:
                        @pl.when(t == 0)
                        def _():
                            for e in range(epg):
                                wb_wait(prev_out, b * epg + e)

                    g = t * _K + b
                    gathers.append(pltpu.async_copy(
                        table.at[idx_v.at[g]],
                        rows_v.at[pl.ds(b * _G, _G)],
                        sem_g.at[b]))

                for b in range(_K):
                    gathers[b].wait()
                    g = t * _K + b
                    for e in range(epg):
                        pltpu.async_copy(
                            rows_v.at[pl.ds(b * _G + e * L, L)],
                            out_hbm.at[base_e + g * epg + e],
                            sem_w.at[b * epg + e])
                return carry

            lax.fori_loop(0, n_outer, outer, 0)

        side(src_t, idx_src_v, src_o, None)
        side(tgt_t, idx_tgt_v, tgt_o, src_o)

        for j in range(_K * epg):
            wb_wait(tgt_o, j)

    return k(src_table, tgt_table, src_idx, tgt_idx)


def kernel(src_table, tgt_table, src_indices, tgt_indices):
    B, L = src_indices.shape
    info = plsc.get_sparse_core_info()
    NW = info.num_cores * info.num_subcores
    src_i = src_indices.astype(jnp.int32).reshape(NW, (B * L) // (NW * _G), _G)
    tgt_i = tgt_indices.astype(jnp.int32).reshape(NW, (B * L) // (NW * _G), _G)
    src_out, tgt_out = _sc_lookup(src_table, tgt_table, src_i, tgt_i, B, L)
    return (src_out, tgt_out)


# speedup vs baseline: 1.0051x; 1.0051x over previous
import functools

import jax
import jax.numpy as jnp
from jax import lax
from jax.experimental import pallas as pl
from jax.experimental.pallas import tpu as pltpu
from jax.experimental.pallas import tpu_sc as plsc

_G = 100
_K = 8


def _sc_lookup(src_table, tgt_table, src_idx, tgt_idx, B, L):
    V, D = src_table.shape
    NW, n_g, _ = src_idx.shape
    per_w = B // NW
    n_outer = n_g // _K
    epg = _G // L

    mesh = plsc.VectorSubcoreMesh(core_axis_name="c", subcore_axis_name="s")
    info = plsc.get_sparse_core_info()
    NC = info.num_cores

    @functools.partial(
        pl.kernel,
        mesh=mesh,
        compiler_params=pltpu.CompilerParams(use_tc_tiling_on_sc=True),
        out_type=[
            jax.ShapeDtypeStruct((B, L, D), jnp.float32),
            jax.ShapeDtypeStruct((B, L, D), jnp.float32),
        ],
        scratch_types=[
            pltpu.VMEM((n_g, _G), jnp.int32),
            pltpu.VMEM((n_g, _G), jnp.int32),
            pltpu.VMEM((_K * _G, D), jnp.float32),
            pltpu.SemaphoreType.DMA((_K,)),
            pltpu.SemaphoreType.DMA((_K * epg,)),
        ],
    )
    def k(src_t, tgt_t, src_i, tgt_i, src_o, tgt_o,
          idx_src_v, idx_tgt_v, rows_v, sem_g, sem_w):
        wid = lax.axis_index("s") * NC + lax.axis_index("c")
        base_e = wid * per_w

        pltpu.sync_copy(src_i.at[wid], idx_src_v)
        pltpu.sync_copy(tgt_i.at[wid], idx_tgt_v)

        def wb_wait(out_hbm, j):
            pltpu.make_async_copy(
                rows_v.at[pl.ds(0, L)], out_hbm.at[base_e], sem_w.at[j]
            ).wait()

        def side(table, idx_v, out_hbm, prev_out):
            def outer(t, carry):
                gathers = []
                for b in range(_K):
                    @pl.when(t > 0)
                    def _():
                        for e in range(epg):
                            wb_wait(out_hbm, b * epg + e)

                    if prev_out is not None:
                        @pl.when(t == 0)
                        def _():
                            for e in range(epg):
                                wb_wait(prev_out, b * epg + e)

                    g = t * _K + b
                    gathers.append(pltpu.async_copy(
                        table.at[idx_v.at[g]],
                        rows_v.at[pl.ds(b * _G, _G)],
                        sem_g.at[b]))

                for b in range(_K):
                    gathers[b].wait()
                    g = t * _K + b
                    for e in range(epg):
                        pltpu.async_copy(
                            rows_v.at[pl.ds(b * _G + e * L, L)],
                            out_hbm.at[base_e + g * epg + e],
                            sem_w.at[b * epg + e])
                return carry

            lax.fori_loop(0, n_outer, outer, 0)

        side(src_t, idx_src_v, src_o, None)
        side(tgt_t, idx_tgt_v, tgt_o, src_o)

        for j in range(_K * epg):
            wb_wait(tgt_o, j)

    return k(src_table, tgt_table, src_idx, tgt_idx)


def kernel(src_table, tgt_table, src_indices, tgt_indices):
    B, L = src_indices.shape
    info = plsc.get_sparse_core_info()
    NW = info.num_cores * info.num_subcores
    src_i = src_indices.astype(jnp.int32).reshape(NW, (B * L) // (NW * _G), _G)
    tgt_i = tgt_indices.astype(jnp.int32).reshape(NW, (B * L) // (NW * _G), _G)
    src_out, tgt_out = _sc_lookup(src_table, tgt_table, src_i, tgt_i, B, L)
    return (src_out, tgt_out)


# speedup vs baseline: 1.8112x; 1.8020x over previous
import functools

import jax
import jax.numpy as jnp
from jax import lax
from jax.experimental import pallas as pl
from jax.experimental.pallas import tpu as pltpu
from jax.experimental.pallas import tpu_sc as plsc

_CHUNK = 128
_K = 5


def _sc_lookup(src_table, tgt_table, src_idx, tgt_idx):
    V, D = src_table.shape
    NW, NCH, _ = src_idx.shape
    N = NW * NCH * _CHUNK
    per_w = NCH * _CHUNK
    n_outer = NCH // _K

    mesh = plsc.VectorSubcoreMesh(core_axis_name="c", subcore_axis_name="s")
    info = plsc.get_sparse_core_info()
    NC = info.num_cores

    @functools.partial(
        pl.kernel,
        mesh=mesh,
        out_type=[
            jax.ShapeDtypeStruct((N, D), jnp.float32),
            jax.ShapeDtypeStruct((N, D), jnp.float32),
        ],
        scratch_types=[
            pltpu.VMEM((NCH, _CHUNK), jnp.int32),
            pltpu.VMEM((NCH, _CHUNK), jnp.int32),
            pltpu.VMEM((_K * _CHUNK, D), jnp.float32),
            pltpu.SemaphoreType.DMA((_K,)),
            pltpu.SemaphoreType.DMA((_K,)),
        ],
    )
    def k(src_t, tgt_t, src_i, tgt_i, src_o, tgt_o,
          idx_src_v, idx_tgt_v, rows_v, sem_g, sem_w):
        wid = lax.axis_index("s") * NC + lax.axis_index("c")
        base = wid * per_w

        pltpu.sync_copy(src_i.at[wid], idx_src_v)
        pltpu.sync_copy(tgt_i.at[wid], idx_tgt_v)

        def wb_wait(out_hbm, b):
            pltpu.make_async_copy(
                rows_v.at[pl.ds(0, _CHUNK)],
                out_hbm.at[pl.ds(base, _CHUNK)],
                sem_w.at[b],
            ).wait()

        def side(table, idx_v, out_hbm, prev_out):
            def outer(t, carry):
                gathers = []
                for b in range(_K):
                    @pl.when(t > 0)
                    def _():
                        wb_wait(out_hbm, b)

                    if prev_out is not None:
                        @pl.when(t == 0)
                        def _():
                            wb_wait(prev_out, b)

                    c = t * _K + b
                    gathers.append(pltpu.async_copy(
                        table.at[idx_v.at[c]],
                        rows_v.at[pl.ds(b * _CHUNK, _CHUNK)],
                        sem_g.at[b]))

                for b in range(_K):
                    gathers[b].wait()
                    c = t * _K + b
                    pltpu.async_copy(
                        rows_v.at[pl.ds(b * _CHUNK, _CHUNK)],
                        out_hbm.at[pl.ds(base + c * _CHUNK, _CHUNK)],
                        sem_w.at[b])
                return carry

            lax.fori_loop(0, n_outer, outer, 0)

        side(src_t, idx_src_v, src_o, None)
        side(tgt_t, idx_tgt_v, tgt_o, src_o)

        for b in range(_K):
            wb_wait(tgt_o, b)

    return k(src_table, tgt_table, src_idx, tgt_idx)


def kernel(src_table, tgt_table, src_indices, tgt_indices):
    B, L = src_indices.shape
    D = src_table.shape[1]
    info = plsc.get_sparse_core_info()
    NW = info.num_cores * info.num_subcores
    NCH = (B * L) // (NW * _CHUNK)
    src_i = src_indices.astype(jnp.int32).T.reshape(NW, NCH, _CHUNK)
    tgt_i = tgt_indices.astype(jnp.int32).T.reshape(NW, NCH, _CHUNK)
    src_out, tgt_out = _sc_lookup(src_table, tgt_table, src_i, tgt_i)
    src_out = src_out.reshape(L, B, D).transpose(1, 0, 2)
    tgt_out = tgt_out.reshape(L, B, D).transpose(1, 0, 2)
    return (src_out, tgt_out)


# speedup vs baseline: 1.8596x; 1.0268x over previous
import functools

import jax
import jax.numpy as jnp
from jax import lax
from jax.experimental import pallas as pl
from jax.experimental.pallas import tpu as pltpu
from jax.experimental.pallas import tpu_sc as plsc

_K = 5


def _sc_lookup(src_table, tgt_table, src_idx, tgt_idx):
    V, D = src_table.shape
    L, B = src_idx.shape
    N = L * B
    n_outer = L // _K

    mesh = plsc.VectorSubcoreMesh(core_axis_name="c", subcore_axis_name="s")
    info = plsc.get_sparse_core_info()
    NC = info.num_cores
    NW = NC * info.num_subcores
    W = B // NW

    @functools.partial(
        pl.kernel,
        mesh=mesh,
        out_type=[
            jax.ShapeDtypeStruct((N, D), jnp.float32),
            jax.ShapeDtypeStruct((N, D), jnp.float32),
        ],
        scratch_types=[
            pltpu.VMEM((L, W), jnp.int32),
            pltpu.VMEM((L, W), jnp.int32),
            pltpu.VMEM((_K * W, D), jnp.float32),
            pltpu.SemaphoreType.DMA((_K,)),
            pltpu.SemaphoreType.DMA((_K,)),
        ],
    )
    def k(src_t, tgt_t, src_i, tgt_i, src_o, tgt_o,
          idx_src_v, idx_tgt_v, rows_v, sem_g, sem_w):
        wid = lax.axis_index("s") * NC + lax.axis_index("c")
        col = wid * W

        pltpu.sync_copy(src_i.at[:, pl.ds(col, W)], idx_src_v)
        pltpu.sync_copy(tgt_i.at[:, pl.ds(col, W)], idx_tgt_v)

        def wb_wait(out_hbm, b):
            pltpu.make_async_copy(
                rows_v.at[pl.ds(0, W)],
                out_hbm.at[pl.ds(col, W)],
                sem_w.at[b],
            ).wait()

        def side(table, idx_v, out_hbm, prev_out):
            def outer(t, carry):
                gathers = []
                for b in range(_K):
                    @pl.when(t > 0)
                    def _():
                        wb_wait(out_hbm, b)

                    if prev_out is not None:
                        @pl.when(t == 0)
                        def _():
                            wb_wait(prev_out, b)

                    l = t * _K + b
                    gathers.append(pltpu.async_copy(
                        table.at[idx_v.at[l]],
                        rows_v.at[pl.ds(b * W, W)],
                        sem_g.at[b]))

                for b in range(_K):
                    gathers[b].wait()
                    l = t * _K + b
                    pltpu.async_copy(
                        rows_v.at[pl.ds(b * W, W)],
                        out_hbm.at[pl.ds(l * B + col, W)],
                        sem_w.at[b])
                return carry

            lax.fori_loop(0, n_outer, outer, 0)

        side(src_t, idx_src_v, src_o, None)
        side(tgt_t, idx_tgt_v, tgt_o, src_o)

        for b in range(_K):
            wb_wait(tgt_o, b)

    return k(src_table, tgt_table, src_idx, tgt_idx)


def kernel(src_table, tgt_table, src_indices, tgt_indices):
    B, L = src_indices.shape
    D = src_table.shape[1]
    src_out, tgt_out = _sc_lookup(
        src_table, tgt_table,
        src_indices.astype(jnp.int32).T, tgt_indices.astype(jnp.int32).T)
    src_out = src_out.reshape(L, B, D).transpose(1, 0, 2)
    tgt_out = tgt_out.reshape(L, B, D).transpose(1, 0, 2)
    return (src_out, tgt_out)
